# baseline (speedup 1.0000x reference)
import functools

import jax
import jax.numpy as jnp
from jax import lax
from jax.experimental import pallas as pl
from jax.experimental.pallas import tpu as pltpu
from jax.experimental.pallas import tpu_sc as plsc

_B = 16384
_D = 16
_NT = 16
_BPT = _B // _NT
_PANEL = 512
_PG = _PANEL // 128

_HEAVY = [(0, 1, 1000000),
          (2, 8, 100000), (3, 9, 100000), (4, 10, 100000),
          (5, 11, 100000), (6, 12, 100000), (7, 13, 100000)]

_SZ_BIG = ((1000000 + _PANEL - 1) // _PANEL + 1) * (_PANEL // 8)
_SZ_BIG = (( (1000000 + 127) // 128 + _PG - 1) // _PG) * (_PANEL // 8)
_SZ_MID = (( (100000 + 127) // 128 + _PG - 1) // _PG) * (_PANEL // 8)
_SCRATCH_ROWS = 2 * _SZ_BIG + 12 * _SZ_MID


def _transpose_panel_cols(panel, blk):
    iota = lax.iota(jnp.int32, 16)

    @plsc.parallel_loop(0, _PANEL // 16, step=1, unroll=4)
    def _(bi):
        l0 = bi * 16
        for k in range(16):
            cdiag = (iota + k) & 15
            gv = plsc.load_gather(panel, [iota, l0 + cdiag])
            rows = l0 // 8 + lax.shift_right_logical(cdiag, 3)
            cols = (cdiag & 7) * 16 + iota
            plsc.store_scatter(blk, [rows, cols], gv)


@functools.partial(
    pl.kernel,
    out_type=(jax.ShapeDtypeStruct((26 * _D, _B), jnp.float32),
              jax.ShapeDtypeStruct((_SCRATCH_ROWS, 128), jnp.float32)),
    mesh=plsc.VectorSubcoreMesh(core_axis_name="c", subcore_axis_name="s"),
    scratch_types=[
        pltpu.VMEM((16, _PANEL), jnp.float32),
        pltpu.VMEM((16, _PANEL), jnp.float32),
        pltpu.VMEM((_PANEL // 8, 128), jnp.float32),
        pltpu.VMEM((_PANEL // 8, 128), jnp.float32),
        pltpu.VMEM((16, 1024), jnp.float32),
        pltpu.VMEM((13 * _BPT,), jnp.int32),
        pltpu.VMEM((_BPT,), jnp.int32),
        pltpu.VMEM((128, 128), jnp.float32),
        pltpu.VMEM((128, 128), jnp.float32),
        pltpu.VMEM((16, _BPT), jnp.float32),
        pltpu.SemaphoreType.DMA,
        pltpu.SemaphoreType.DMA,
        pltpu.SemaphoreType.DMA,
        pltpu.SemaphoreType.DMA,
        pltpu.SemaphoreType.DMA,
        pltpu.SemaphoreType.DMA,
        pltpu.SemaphoreType.DMA,
    ],
    compiler_params=pltpu.CompilerParams(
        use_tc_tiling_on_sc=True, needs_layout_passes=False),
)
def _cat_embed(idx_flat, wsmall, *refs):
    wts = refs[:14]
    out = refs[14]
    scratch = refs[15]
    (pan0, pan1, blk0, blk1, tab, idxall, rowsb, gbuf, gbuf2, fb,
     sem0, sem1, gsem, gsem2, isem, bsem0, bsem1) = refs[16:]

    cid = lax.axis_index("c")
    sid = lax.axis_index("s")
    iota = lax.iota(jnp.int32, 16)
    b0 = pl.multiple_of(sid * _BPT, 128)

    for k in range(7):
        jj = jnp.where(cid == 0,
                       jnp.where(k == 0, 0, k + 1),
                       jnp.where(k == 0, 1, k + 7))
        pltpu.async_copy(idx_flat.at[pl.ds(pl.multiple_of(jj * _B + b0, 8), _BPT)],
                         idxall.at[pl.ds(k * _BPT, _BPT)], isem)
    for k in range(6):
        jj = jnp.where(cid == 0, 14 + k, 20 + k)
        pltpu.async_copy(idx_flat.at[pl.ds(pl.multiple_of(jj * _B + b0, 8), _BPT)],
                         idxall.at[pl.ds((7 + k) * _BPT, _BPT)], isem)

    _RPP = _PANEL // 8
    _NPF_BIG = _SZ_BIG // _RPP
    _NPF_MID = _SZ_MID // _RPP
    _NPBIG = (_NPF_BIG + _NT - 1) // _NT
    _NPMID = (_NPF_MID + _NT - 1) // _NT
    _UTOT = _NPBIG + 6 * _NPMID

    def _u2kt(u):
        kbig = u < _NPBIG
        k = jnp.where(kbig, 0, 1 + (u - _NPBIG) // _NPMID)
        t = jnp.where(kbig, u, (u - _NPBIG) % _NPMID)
        npf = jnp.where(kbig, _NPF_BIG, _NPF_MID)
        rbase = (jnp.where(kbig, 0, 2 * _SZ_BIG + (k - 1) * 2 * _SZ_MID)
                 + cid * jnp.where(kbig, _SZ_BIG, _SZ_MID))
        pt = t * _NT + sid
        live = (pt < npf) & (u < _UTOT)
        return k, pt, live, rbase

    def issue(u, buf, sem):
        k, pt, live, _ = _u2kt(u)
        pcol = pl.multiple_of(pt * _PANEL, 128)
        for kk, (ja, jb, c) in enumerate(_HEAVY):
            @pl.when(live & (k == kk) & (cid == 0))
            def _():
                pltpu.async_copy(wts[ja].at[:, pl.ds(pcol, _PANEL)], buf, sem)

            @pl.when(live & (k == kk) & (cid == 1))
            def _():
                pltpu.async_copy(wts[jb].at[:, pl.ds(pcol, _PANEL)], buf, sem)

    def wait_compute(u, buf, sem, blkb, bsem):
        k, pt, live, rbase = _u2kt(u)
        _, _, live_prev, _ = _u2kt(u - 2)

        @pl.when((u >= 2) & live_prev)
        def _():
            pltpu.make_async_copy(scratch.at[pl.ds(0, _RPP), :], blkb, bsem).wait()

        @pl.when(live)
        def _():
            pltpu.make_async_copy(wts[0].at[:, pl.ds(0, _PANEL)], buf, sem).wait()
            _transpose_panel_cols(buf, blkb)
            r0 = pl.multiple_of(pt * _RPP, 8)
            pltpu.async_copy(blkb, scratch.at[pl.ds(rbase + r0, _RPP), :], bsem)

    issue(jnp.int32(0), pan0, sem0)
    issue(jnp.int32(1), pan1, sem1)

    def t_body(i, carry):
        u0 = i * 2
        wait_compute(u0, pan0, sem0, blk0, bsem0)
        issue(u0 + 2, pan0, sem0)
        wait_compute(u0 + 1, pan1, sem1, blk1, bsem1)
        issue(u0 + 3, pan1, sem1)
        return carry

    lax.fori_loop(0, (_UTOT + 1) // 2, t_body, 0)

    for uu in (_UTOT - 1, _UTOT):
        blkb, bsem = (blk0, bsem0) if uu % 2 == 0 else (blk1, bsem1)
        _, _, lv, _ = _u2kt(jnp.int32(uu))

        @pl.when(lv)
        def _():
            pltpu.make_async_copy(scratch.at[pl.ds(0, _RPP), :], blkb, bsem).wait()

    plsc.subcore_barrier()
    pltpu.make_async_copy(idx_flat.at[pl.ds(0, 13 * _BPT)], idxall, isem).wait()

    def heavy_body(k, carry):
        j = jnp.where(cid == 0,
                      jnp.where(k == 0, 0, k + 1),
                      jnp.where(k == 0, 1, k + 7))
        sz = jnp.where(k == 0, _SZ_BIG, _SZ_MID)
        rbase = (jnp.where(k == 0, 0, 2 * _SZ_BIG + (k - 1) * 2 * _SZ_MID)
                 + cid * sz)

        ibase = k * _BPT

        @plsc.parallel_loop(0, _BPT // 16, step=1, unroll=4)
        def _(i):
            v = idxall[pl.ds(ibase + i * 16, 16)]
            rowsb[pl.ds(i * 16, 16)] = rbase + lax.shift_right_logical(v, 3)

        def extract(h, buf):
            @plsc.parallel_loop(0, 8, step=1, unroll=4)
            def _(kk):
                v = idxall[pl.ds(ibase + h * 128 + kk * 16, 16)]
                q16 = (v & 7) * 16
                b16 = iota + kk * 16
                bcols = h * 128 + b16
                for s in range(16):
                    drows = (iota + s) & 15
                    gv = plsc.load_gather(buf, [b16, q16 + drows])
                    plsc.store_scatter(fb, [drows, bcols], gv)

        def issue_g(h, buf, sem):
            @pl.when(h < _BPT // 128)
            def _():
                pltpu.async_copy(scratch.at[rowsb.at[pl.ds(h * 128, 128)]],
                                 buf, sem)

        def wait_g(buf, sem):
            pltpu.make_async_copy(scratch.at[pl.ds(0, 128), :], buf, sem).wait()

        issue_g(jnp.int32(0), gbuf, gsem)
        issue_g(jnp.int32(1), gbuf2, gsem2)

        def g_body(i, carry2):
            h0 = i * 2
            wait_g(gbuf, gsem)
            extract(h0, gbuf)
            issue_g(h0 + 2, gbuf, gsem)
            wait_g(gbuf2, gsem2)
            extract(h0 + 1, gbuf2)
            issue_g(h0 + 3, gbuf2, gsem2)
            return carry2

        lax.fori_loop(0, _BPT // 256, g_body, 0)

        row0 = pl.multiple_of(j * _D, 8)
        pltpu.sync_copy(fb, out.at[pl.ds(row0, _D), pl.ds(b0, _BPT)])
        return carry

    lax.fori_loop(0, 7, heavy_body, 0)

    def small_body(k, carry):
        j = jnp.where(cid == 0, 14 + k, 20 + k)
        tcol = pl.multiple_of((j - 14) * 1024, 128)
        pltpu.sync_copy(wsmall.at[:, pl.ds(tcol, 1024)], tab)
        ibase = (7 + k) * _BPT

        @plsc.parallel_loop(0, _BPT // 16, step=1, unroll=4)
        def _(i):
            cols = idxall[pl.ds(ibase + i * 16, 16)]
            for d in range(16):
                fb[d, pl.ds(i * 16, 16)] = plsc.load_gather(
                    tab, [jnp.full((16,), d, jnp.int32), cols])

        row0 = pl.multiple_of(j * _D, 8)
        pltpu.sync_copy(fb, out.at[pl.ds(row0, _D), pl.ds(b0, _BPT)])
        return carry

    lax.fori_loop(0, 6, small_body, 0)


def kernel(f00, W_f00, f01, W_f01, f02, W_f02, f03, W_f03, f04, W_f04,
           f05, W_f05, f06, W_f06, f07, W_f07, f08, W_f08, f09, W_f09,
           f10, W_f10, f11, W_f11, f12, W_f12, f13, W_f13, f14, W_f14,
           f15, W_f15, f16, W_f16, f17, W_f17, f18, W_f18, f19, W_f19,
           f20, W_f20, f21, W_f21, f22, W_f22, f23, W_f23, f24, W_f24,
           f25, W_f25):
    raw = (f00, W_f00, f01, W_f01, f02, W_f02, f03, W_f03, f04, W_f04,
           f05, W_f05, f06, W_f06, f07, W_f07, f08, W_f08, f09, W_f09,
           f10, W_f10, f11, W_f11, f12, W_f12, f13, W_f13, f14, W_f14,
           f15, W_f15, f16, W_f16, f17, W_f17, f18, W_f18, f19, W_f19,
           f20, W_f20, f21, W_f21, f22, W_f22, f23, W_f23, f24, W_f24,
           f25, W_f25)
    idx_flat = jnp.concatenate([raw[2 * j] for j in range(26)])
    wsmall = jnp.concatenate(
        [jnp.pad(raw[2 * j + 1].T, ((0, 0), (0, 24))) for j in range(14, 26)],
        axis=1)
    heavy = [raw[2 * j + 1].T for j in range(14)]
    out_t, _ = _cat_embed(idx_flat, wsmall, *heavy)
    return out_t.T


# speedup vs baseline: 1.0052x; 1.0052x over previous
import functools

import jax
import jax.numpy as jnp
from jax import lax
from jax.experimental import pallas as pl
from jax.experimental.pallas import tpu as pltpu
from jax.experimental.pallas import tpu_sc as plsc

_B = 16384
_D = 16
_NT = 16
_BPT = _B // _NT
_PANEL = 512
_PG = _PANEL // 128

_HEAVY = [(0, 1, 1000000),
          (2, 8, 100000), (3, 9, 100000), (4, 10, 100000),
          (5, 11, 100000), (6, 12, 100000), (7, 13, 100000)]

_SZ_BIG = ((1000000 + _PANEL - 1) // _PANEL + 1) * (_PANEL // 8)
_SZ_BIG = (( (1000000 + 127) // 128 + _PG - 1) // _PG) * (_PANEL // 8)
_SZ_MID = (( (100000 + 127) // 128 + _PG - 1) // _PG) * (_PANEL // 8)
_SCRATCH_ROWS = 2 * _SZ_BIG + 12 * _SZ_MID


def _transpose_panel_cols(panel, blk):
    iota = lax.iota(jnp.int32, 16)

    @plsc.parallel_loop(0, _PANEL // 16, step=1, unroll=4)
    def _(bi):
        l0 = bi * 16
        for k in range(16):
            cdiag = (iota + k) & 15
            gv = plsc.load_gather(panel, [iota, l0 + cdiag])
            rows = l0 // 8 + lax.shift_right_logical(cdiag, 3)
            cols = (cdiag & 7) * 16 + iota
            plsc.store_scatter(blk, [rows, cols], gv)


@functools.partial(
    pl.kernel,
    out_type=(jax.ShapeDtypeStruct((26 * _D, _B), jnp.float32),
              jax.ShapeDtypeStruct((_SCRATCH_ROWS, 128), jnp.float32)),
    mesh=plsc.VectorSubcoreMesh(core_axis_name="c", subcore_axis_name="s"),
    scratch_types=[
        pltpu.VMEM((16, _PANEL), jnp.float32),
        pltpu.VMEM((16, _PANEL), jnp.float32),
        pltpu.VMEM((_PANEL // 8, 128), jnp.float32),
        pltpu.VMEM((_PANEL // 8, 128), jnp.float32),
        pltpu.VMEM((16, 1024), jnp.float32),
        pltpu.VMEM((13 * _BPT,), jnp.int32),
        pltpu.VMEM((_BPT,), jnp.int32),
        pltpu.VMEM((128, 128), jnp.float32),
        pltpu.VMEM((128, 128), jnp.float32),
        pltpu.VMEM((16, _BPT), jnp.float32),
        pltpu.SemaphoreType.DMA,
        pltpu.SemaphoreType.DMA,
        pltpu.SemaphoreType.DMA,
        pltpu.SemaphoreType.DMA,
        pltpu.SemaphoreType.DMA,
        pltpu.SemaphoreType.DMA,
        pltpu.SemaphoreType.DMA,
    ],
    compiler_params=pltpu.CompilerParams(
        use_tc_tiling_on_sc=True, needs_layout_passes=False),
)
def _cat_embed(idx_flat, wsmall, *refs):
    wts = refs[:14]
    out = refs[14]
    scratch = refs[15]
    (pan0, pan1, blk0, blk1, tab, idxall, rowsb, gbuf, gbuf2, fb,
     sem0, sem1, gsem, gsem2, isem, bsem0, bsem1) = refs[16:]

    cid = lax.axis_index("c")
    sid = lax.axis_index("s")
    iota = lax.iota(jnp.int32, 16)
    b0 = pl.multiple_of(sid * _BPT, 128)

    for k in range(7):
        jj = jnp.where(cid == 0,
                       jnp.where(k == 0, 0, k + 1),
                       jnp.where(k == 0, 1, k + 7))
        pltpu.async_copy(idx_flat.at[pl.ds(pl.multiple_of(jj * _B + b0, 8), _BPT)],
                         idxall.at[pl.ds(k * _BPT, _BPT)], isem)
    for k in range(6):
        jj = jnp.where(cid == 0, 14 + k, 20 + k)
        pltpu.async_copy(idx_flat.at[pl.ds(pl.multiple_of(jj * _B + b0, 8), _BPT)],
                         idxall.at[pl.ds((7 + k) * _BPT, _BPT)], isem)

    _RPP = _PANEL // 8
    _NPF_BIG = _SZ_BIG // _RPP
    _NPF_MID = _SZ_MID // _RPP
    _NPBIG = (_NPF_BIG + _NT - 1) // _NT
    _NPMID = (_NPF_MID + _NT - 1) // _NT
    _UTOT = _NPBIG + 6 * _NPMID

    def _u2kt(u):
        kbig = u < _NPBIG
        k = jnp.where(kbig, 0, 1 + (u - _NPBIG) // _NPMID)
        t = jnp.where(kbig, u, (u - _NPBIG) % _NPMID)
        npf = jnp.where(kbig, _NPF_BIG, _NPF_MID)
        rbase = (jnp.where(kbig, 0, 2 * _SZ_BIG + (k - 1) * 2 * _SZ_MID)
                 + cid * jnp.where(kbig, _SZ_BIG, _SZ_MID))
        pt = t * _NT + sid
        live = (pt < npf) & (u < _UTOT)
        return k, pt, live, rbase

    def issue(u, buf, sem):
        k, pt, live, _ = _u2kt(u)
        pcol = pl.multiple_of(pt * _PANEL, 128)
        for kk, (ja, jb, c) in enumerate(_HEAVY):
            @pl.when(live & (k == kk) & (cid == 0))
            def _():
                pltpu.async_copy(wts[ja].at[:, pl.ds(pcol, _PANEL)], buf, sem)

            @pl.when(live & (k == kk) & (cid == 1))
            def _():
                pltpu.async_copy(wts[jb].at[:, pl.ds(pcol, _PANEL)], buf, sem)

    def wait_compute(u, buf, sem, blkb, bsem):
        k, pt, live, rbase = _u2kt(u)
        _, _, live_prev, _ = _u2kt(u - 2)

        @pl.when((u >= 2) & live_prev)
        def _():
            pltpu.make_async_copy(scratch.at[pl.ds(0, _RPP), :], blkb, bsem).wait()

        @pl.when(live)
        def _():
            pltpu.make_async_copy(wts[0].at[:, pl.ds(0, _PANEL)], buf, sem).wait()
            _transpose_panel_cols(buf, blkb)
            r0 = pl.multiple_of(pt * _RPP, 8)
            pltpu.async_copy(blkb, scratch.at[pl.ds(rbase + r0, _RPP), :], bsem)

    issue(jnp.int32(0), pan0, sem0)
    issue(jnp.int32(1), pan1, sem1)

    def t_body(i, carry):
        u0 = i * 2
        wait_compute(u0, pan0, sem0, blk0, bsem0)
        issue(u0 + 2, pan0, sem0)
        wait_compute(u0 + 1, pan1, sem1, blk1, bsem1)
        issue(u0 + 3, pan1, sem1)
        return carry

    lax.fori_loop(0, (_UTOT + 1) // 2, t_body, 0)

    for uu in (_UTOT - 1, _UTOT):
        blkb, bsem = (blk0, bsem0) if uu % 2 == 0 else (blk1, bsem1)
        _, _, lv, _ = _u2kt(jnp.int32(uu))

        @pl.when(lv)
        def _():
            pltpu.make_async_copy(scratch.at[pl.ds(0, _RPP), :], blkb, bsem).wait()

    plsc.subcore_barrier()
    pltpu.make_async_copy(idx_flat.at[pl.ds(0, 13 * _BPT)], idxall, isem).wait()

    def heavy_body(k, carry):
        j = jnp.where(cid == 0,
                      jnp.where(k == 0, 0, k + 1),
                      jnp.where(k == 0, 1, k + 7))
        sz = jnp.where(k == 0, _SZ_BIG, _SZ_MID)
        rbase = (jnp.where(k == 0, 0, 2 * _SZ_BIG + (k - 1) * 2 * _SZ_MID)
                 + cid * sz)

        ibase = k * _BPT

        @plsc.parallel_loop(0, _BPT // 16, step=1, unroll=4)
        def _(i):
            v = idxall[pl.ds(ibase + i * 16, 16)]
            rowsb[pl.ds(i * 16, 16)] = rbase + lax.shift_right_logical(v, 3)

        def extract(h, buf):
            @plsc.parallel_loop(0, 8, step=1, unroll=2)
            def _(kk):
                v = idxall[pl.ds(ibase + h * 128 + kk * 16, 16)]
                q16 = (v & 7) * 16
                b16 = iota + kk * 16
                bcols = h * 128 + b16
                for s in range(16):
                    drows = (iota + s) & 15
                    gv = plsc.load_gather(buf, [b16, q16 + drows])
                    plsc.store_scatter(fb, [drows, bcols], gv)

        def issue_g(h, buf, sem):
            @pl.when(h < _BPT // 128)
            def _():
                pltpu.async_copy(scratch.at[rowsb.at[pl.ds(h * 128, 128)]],
                                 buf, sem)

        def wait_g(buf, sem):
            pltpu.make_async_copy(scratch.at[pl.ds(0, 128), :], buf, sem).wait()

        issue_g(jnp.int32(0), gbuf, gsem)
        issue_g(jnp.int32(1), gbuf2, gsem2)

        def g_body(i, carry2):
            h0 = i * 2
            wait_g(gbuf, gsem)
            extract(h0, gbuf)
            issue_g(h0 + 2, gbuf, gsem)
            wait_g(gbuf2, gsem2)
            extract(h0 + 1, gbuf2)
            issue_g(h0 + 3, gbuf2, gsem2)
            return carry2

        lax.fori_loop(0, _BPT // 256, g_body, 0)

        row0 = pl.multiple_of(j * _D, 8)
        pltpu.sync_copy(fb, out.at[pl.ds(row0, _D), pl.ds(b0, _BPT)])
        return carry

    lax.fori_loop(0, 7, heavy_body, 0)

    def small_body(k, carry):
        j = jnp.where(cid == 0, 14 + k, 20 + k)
        tcol = pl.multiple_of((j - 14) * 1024, 128)
        pltpu.sync_copy(wsmall.at[:, pl.ds(tcol, 1024)], tab)
        ibase = (7 + k) * _BPT

        @plsc.parallel_loop(0, _BPT // 16, step=1, unroll=2)
        def _(i):
            cols = idxall[pl.ds(ibase + i * 16, 16)]
            for d in range(16):
                fb[d, pl.ds(i * 16, 16)] = plsc.load_gather(
                    tab, [jnp.full((16,), d, jnp.int32), cols])

        row0 = pl.multiple_of(j * _D, 8)
        pltpu.sync_copy(fb, out.at[pl.ds(row0, _D), pl.ds(b0, _BPT)])
        return carry

    lax.fori_loop(0, 6, small_body, 0)


def kernel(f00, W_f00, f01, W_f01, f02, W_f02, f03, W_f03, f04, W_f04,
           f05, W_f05, f06, W_f06, f07, W_f07, f08, W_f08, f09, W_f09,
           f10, W_f10, f11, W_f11, f12, W_f12, f13, W_f13, f14, W_f14,
           f15, W_f15, f16, W_f16, f17, W_f17, f18, W_f18, f19, W_f19,
           f20, W_f20, f21, W_f21, f22, W_f22, f23, W_f23, f24, W_f24,
           f25, W_f25):
    raw = (f00, W_f00, f01, W_f01, f02, W_f02, f03, W_f03, f04, W_f04,
           f05, W_f05, f06, W_f06, f07, W_f07, f08, W_f08, f09, W_f09,
           f10, W_f10, f11, W_f11, f12, W_f12, f13, W_f13, f14, W_f14,
           f15, W_f15, f16, W_f16, f17, W_f17, f18, W_f18, f19, W_f19,
           f20, W_f20, f21, W_f21, f22, W_f22, f23, W_f23, f24, W_f24,
           f25, W_f25)
    idx_flat = jnp.concatenate([raw[2 * j] for j in range(26)])
    wsmall = jnp.concatenate(
        [jnp.pad(raw[2 * j + 1].T, ((0, 0), (0, 24))) for j in range(14, 26)],
        axis=1)
    heavy = [raw[2 * j + 1].T for j in range(14)]
    out_t, _ = _cat_embed(idx_flat, wsmall, *heavy)
    return out_t.T
